# Initial kernel scaffold; baseline (speedup 1.0000x reference)
import jax
import jax.numpy as jnp
from jax.experimental import pallas as pl


def kernel(x, edge_index, W1, b1, W2, b2, Wf, bf):
    raise NotImplementedError("write your pallas kernel here")


# speedup vs baseline: 8.5485x; 8.5485x over previous
import functools

import jax
import jax.numpy as jnp
from jax import lax
from jax.experimental import pallas as pl
from jax.experimental.pallas import tpu as pltpu
from jax.experimental.pallas import tpu_sc as plsc

N = 10000
E = 160000
LANES = 128
NB = 80
HB = NB // 2
NT_ROWS = 16 * NB
EPAD = NT_ROWS * LANES
DISCARD = N
R = 10240
TPR = R // 16

_f32 = jnp.float32
_MESH = plsc.VectorSubcoreMesh(core_axis_name="c", subcore_axis_name="s")


def _deg_body(dst_hbm, out_hbm, didx, ones_v, zbuf, wbuf, sem, accd):
    c = lax.axis_index("c")
    w = lax.axis_index("s")

    @pl.when(c == 0)
    def _():
        zero16 = jnp.zeros((16,), _f32)
        one16 = jnp.ones((16,), _f32)

        def fill_z(i, _):
            zbuf[pl.ds(i * 16, 16)] = zero16
            return 0

        lax.fori_loop(0, TPR // 16, fill_z, 0)

        def fill_o(i, _):
            ones_v[pl.ds(i * 16, 16)] = one16
            return 0

        lax.fori_loop(0, LANES // 16, fill_o, 0)
        pltpu.sync_copy(dst_hbm.at[pl.ds(w * NB, NB)], didx)
        pltpu.sync_copy(zbuf, accd.at[pl.ds(w * TPR, TPR)])
        plsc.subcore_barrier()

        def body(b, _):
            pltpu.sync_copy(ones_v, accd.at[didx.at[b]], add=True)
            return 0

        lax.fori_loop(0, NB, body, 0)
        plsc.subcore_barrier()

        @pl.when(w == 0)
        def _():
            pltpu.sync_copy(accd, wbuf)
            pltpu.sync_copy(wbuf, out_hbm)


_deg_call = functools.partial(
    pl.kernel,
    out_type=jax.ShapeDtypeStruct((R,), _f32),
    mesh=_MESH,
    scratch_types=[
        pltpu.VMEM((NB, LANES), jnp.int32),
        pltpu.VMEM((LANES,), _f32),
        pltpu.VMEM((TPR,), _f32),
        pltpu.VMEM((R,), _f32),
        pltpu.SemaphoreType.DMA,
        pltpu.VMEM_SHARED((R,), _f32),
    ],
)(_deg_body)


def _chunk_pass(y_hbm, out_hbm, src_hbm, dst_hbm, sidx, didx, rows_a, rows_b,
                sem_a, sem_b, acc, w):
    for j in range(TPR // LANES):
        pltpu.sync_copy(y_hbm.at[pl.ds(w * TPR + j * LANES, LANES)], rows_a)
        pltpu.sync_copy(rows_a, acc.at[pl.ds(w * TPR + j * LANES, LANES)])
    plsc.subcore_barrier()

    for p in range(NB // HB):
        pltpu.sync_copy(src_hbm.at[pl.ds(w * NB + p * HB, HB)], sidx)
        pltpu.sync_copy(dst_hbm.at[pl.ds(w * NB + p * HB, HB)], didx)
        pltpu.async_copy(y_hbm.at[sidx.at[0]], rows_a, sem_a)

        def body(i, _):
            b = 2 * i
            pltpu.make_async_copy(y_hbm.at[sidx.at[b]], rows_a, sem_a).wait()
            pltpu.async_copy(y_hbm.at[sidx.at[b + 1]], rows_b, sem_b)
            pltpu.sync_copy(rows_a, acc.at[didx.at[b]], add=True)
            pltpu.make_async_copy(y_hbm.at[sidx.at[b + 1]], rows_b,
                                  sem_b).wait()

            @pl.when(i < HB // 2 - 1)
            def _():
                pltpu.async_copy(y_hbm.at[sidx.at[b + 2]], rows_a, sem_a)

            pltpu.sync_copy(rows_b, acc.at[didx.at[b + 1]], add=True)
            return 0

        lax.fori_loop(0, HB // 2, body, 0)

    plsc.subcore_barrier()
    for j in range(TPR // LANES):
        pltpu.sync_copy(acc.at[pl.ds(w * TPR + j * LANES, LANES)], rows_a)
        pltpu.sync_copy(rows_a, out_hbm.at[pl.ds(w * TPR + j * LANES, LANES)])


def _make_agg_body(chunks_per_core):
    nc = chunks_per_core

    def body(*refs):
        ys = refs[0:2 * nc]
        src_hbm = refs[2 * nc]
        dst_hbm = refs[2 * nc + 1]
        outs = refs[2 * nc + 2:4 * nc + 2]
        sidx, didx, rows_a, rows_b, sem_a, sem_b, acc = refs[4 * nc + 2:]
        c = lax.axis_index("c")
        w = lax.axis_index("s")

        @pl.when(c == 0)
        def _():
            for k in range(nc):
                _chunk_pass(ys[k], outs[k], src_hbm, dst_hbm, sidx, didx,
                            rows_a, rows_b, sem_a, sem_b, acc, w)

        @pl.when(c == 1)
        def _():
            for k in range(nc, 2 * nc):
                _chunk_pass(ys[k], outs[k], src_hbm, dst_hbm, sidx, didx,
                            rows_a, rows_b, sem_a, sem_b, acc, w)

    return body


_AGG_SCRATCH = [
    pltpu.VMEM((HB, LANES), jnp.int32),
    pltpu.VMEM((HB, LANES), jnp.int32),
    pltpu.VMEM((LANES, 128), _f32),
    pltpu.VMEM((LANES, 128), _f32),
    pltpu.SemaphoreType.DMA,
    pltpu.SemaphoreType.DMA,
    pltpu.VMEM_SHARED((R, 128), _f32),
]

_agg2_call = functools.partial(
    pl.kernel,
    out_type=tuple(jax.ShapeDtypeStruct((R, 128), _f32) for _ in range(2)),
    mesh=_MESH,
    scratch_types=_AGG_SCRATCH,
)(_make_agg_body(1))

_agg4_call = functools.partial(
    pl.kernel,
    out_type=tuple(jax.ShapeDtypeStruct((R, 128), _f32) for _ in range(4)),
    mesh=_MESH,
    scratch_types=_AGG_SCRATCH,
)(_make_agg_body(2))


_BLK = 1000


def _prep_body(deg_ref, x_ref, dinv_ref, ya_ref, yb_ref):
    deg = deg_ref[...] + 1.0
    dinv = lax.rsqrt(deg)
    y = x_ref[...] * dinv
    dinv_ref[...] = dinv
    ya_ref[...] = y[:, :128]
    yb_ref[...] = y[:, 128:]


_prep_call = pl.pallas_call(
    _prep_body,
    grid=(N // _BLK,),
    in_specs=[
        pl.BlockSpec((_BLK, 1), lambda i: (i, 0)),
        pl.BlockSpec((_BLK, 256), lambda i: (i, 0)),
    ],
    out_specs=[
        pl.BlockSpec((_BLK, 1), lambda i: (i, 0)),
        pl.BlockSpec((_BLK, 128), lambda i: (i, 0)),
        pl.BlockSpec((_BLK, 128), lambda i: (i, 0)),
    ],
    out_shape=[
        jax.ShapeDtypeStruct((N, 1), _f32),
        jax.ShapeDtypeStruct((R, 128), _f32),
        jax.ShapeDtypeStruct((R, 128), _f32),
    ],
)


def _mm1_body(sa, sb, dinv, w1, b1, y0, y1, y2, y3):
    dv = dinv[...]
    a = jnp.concatenate([sa[...] * dv, sb[...] * dv], axis=1)
    h = jnp.dot(a, w1[...], preferred_element_type=_f32) + b1[...]
    h = jnp.maximum(h, 0.0) * dv
    y0[...] = h[:, 0:128]
    y1[...] = h[:, 128:256]
    y2[...] = h[:, 256:384]
    y3[...] = h[:, 384:512]


_mm1_call = pl.pallas_call(
    _mm1_body,
    grid=(N // _BLK,),
    in_specs=[
        pl.BlockSpec((_BLK, 128), lambda i: (i, 0)),
        pl.BlockSpec((_BLK, 128), lambda i: (i, 0)),
        pl.BlockSpec((_BLK, 1), lambda i: (i, 0)),
        pl.BlockSpec((256, 512), lambda i: (0, 0)),
        pl.BlockSpec((1, 512), lambda i: (0, 0)),
    ],
    out_specs=[pl.BlockSpec((_BLK, 128), lambda i: (i, 0))] * 4,
    out_shape=[jax.ShapeDtypeStruct((R, 128), _f32)] * 4,
)


def _mm2_body(s0, s1, s2, s3, dinv, w2, b2, wf, bf, out):
    dv = dinv[...]
    a = jnp.concatenate(
        [s0[...] * dv, s1[...] * dv, s2[...] * dv, s3[...] * dv], axis=1)
    h = jnp.dot(a, w2[...], preferred_element_type=_f32) + b2[...]
    h = jnp.maximum(h, 0.0)
    out[...] = jnp.dot(h, wf[...], preferred_element_type=_f32) + bf[...]


_mm2_call = pl.pallas_call(
    _mm2_body,
    grid=(N // _BLK,),
    in_specs=[
        pl.BlockSpec((_BLK, 128), lambda i: (i, 0)),
        pl.BlockSpec((_BLK, 128), lambda i: (i, 0)),
        pl.BlockSpec((_BLK, 128), lambda i: (i, 0)),
        pl.BlockSpec((_BLK, 128), lambda i: (i, 0)),
        pl.BlockSpec((_BLK, 1), lambda i: (i, 0)),
        pl.BlockSpec((512, 512), lambda i: (0, 0)),
        pl.BlockSpec((1, 512), lambda i: (0, 0)),
        pl.BlockSpec((512, 128), lambda i: (0, 0)),
        pl.BlockSpec((1, 128), lambda i: (0, 0)),
    ],
    out_specs=pl.BlockSpec((_BLK, 128), lambda i: (i, 0)),
    out_shape=jax.ShapeDtypeStruct((N, 128), _f32),
)


def kernel(x, edge_index, W1, b1, W2, b2, Wf, bf):
    ei = edge_index.astype(jnp.int32)
    src = jnp.concatenate(
        [ei[0], jnp.zeros((EPAD - E,), jnp.int32)]).reshape(NT_ROWS, LANES)
    dst = jnp.concatenate(
        [ei[1], jnp.full((EPAD - E,), DISCARD, jnp.int32)]).reshape(
            NT_ROWS, LANES)

    deg = _deg_call(dst)[:N].reshape(N, 1)
    dinv, y1a, y1b = _prep_call(deg, x)
    s1a, s1b = _agg2_call(y1a, y1b, src, dst)
    y2 = _mm1_call(s1a, s1b, dinv, W1, b1.reshape(1, -1))
    s2 = _agg4_call(y2[0], y2[1], y2[2], y2[3], src, dst)
    out = _mm2_call(s2[0], s2[1], s2[2], s2[3], dinv, W2,
                    b2.reshape(1, -1), Wf, bf.reshape(1, -1))
    return out
